# baseline (speedup 1.0000x reference)
import functools

import jax
import jax.numpy as jnp
from jax import lax
from jax.experimental import pallas as pl
from jax.experimental.pallas import tpu as pltpu
from jax.experimental.pallas import tpu_sc as plsc

_NC = 2
_NS = 16
_NW = _NC * _NS
_LANES = 16


@functools.cache
def _build_sc_gather(S, D0, D1, D2):
    b_per_w = S // _NW
    C = min(b_per_w, 16384)
    n_chunks = b_per_w // C
    s0 = D1 * D2
    s1 = D2

    mesh = plsc.VectorSubcoreMesh(core_axis_name="c", subcore_axis_name="s")

    @functools.partial(
        pl.kernel,
        mesh=mesh,
        out_type=jax.ShapeDtypeStruct((S,), jnp.float32),
        scratch_types=[
            pltpu.VMEM((C,), jnp.int32),
            pltpu.VMEM((C,), jnp.int32),
            pltpu.VMEM((C,), jnp.int32),
            pltpu.VMEM((C,), jnp.int32),
            pltpu.VMEM((C,), jnp.float32),
            pltpu.SemaphoreType.DMA,
        ],
    )
    def sc_gather(x0_hbm, x1_hbm, x2_hbm, w_hbm, out_hbm,
                  x0_v, x1_v, x2_v, idx_v, out_v, sem):
        wid = lax.axis_index("s") * _NC + lax.axis_index("c")
        base = wid * b_per_w

        def chunk_body(ci, carry):
            off = base + ci * C
            pltpu.sync_copy(x0_hbm.at[pl.ds(off, C)], x0_v)
            pltpu.sync_copy(x1_hbm.at[pl.ds(off, C)], x1_v)
            pltpu.sync_copy(x2_hbm.at[pl.ds(off, C)], x2_v)

            def vec_body(i, c):
                sl = pl.ds(i * _LANES, _LANES)
                idx_v[sl] = x0_v[sl] * s0 + x1_v[sl] * s1 + x2_v[sl]
                return c

            lax.fori_loop(0, C // _LANES, vec_body, 0, unroll=8)
            pltpu.async_copy(w_hbm.at[idx_v], out_v, sem).wait()
            pltpu.sync_copy(out_v, out_hbm.at[pl.ds(off, C)])
            return carry

        lax.fori_loop(0, n_chunks, chunk_body, 0)

    return sc_gather


def kernel(x, weights):
    S = x.shape[0]
    D0, D1, D2 = weights.shape
    x0 = x[:, 0]
    x1 = x[:, 1]
    x2 = x[:, 2]
    w_flat = weights.reshape(D0 * D1 * D2)
    return _build_sc_gather(S, D0, D1, D2)(x0, x1, x2, w_flat)
